# baseline (speedup 1.0000x reference)
import functools

import jax
import jax.numpy as jnp
from jax.experimental import pallas as pl


def _mlp_chain_body(nlayers, relus, x_ref, *refs):
    out_ref = refs[-1]
    x = x_ref[...]
    for i in range(nlayers):
        w = refs[2 * i][...]
        b = refs[2 * i + 1][...]
        x = jax.lax.dot_general(x, w, (((1,), (0,)), ((), ())),
                                preferred_element_type=jnp.float32)
        x = x + b
        if relus[i]:
            x = jnp.maximum(x, 0.0)
    out_ref[...] = x


def _mlp_pallas(x, layers, relus=None, row_tile=512):
    nlayers = len(layers)
    if relus is None:
        relus = (True,) * nlayers
    R, cin = x.shape
    cout = layers[-1][0].shape[1]
    rt = min(row_tile, R)
    Rp = ((R + rt - 1) // rt) * rt
    if Rp != R:
        x = jnp.pad(x, ((0, Rp - R), (0, 0)))
    grid = (Rp // rt,)
    in_specs = [pl.BlockSpec((rt, cin), lambda i: (i, 0))]
    args = [x]
    for (w, b) in layers:
        in_specs.append(pl.BlockSpec(w.shape, lambda i: (0, 0)))
        in_specs.append(pl.BlockSpec((1, b.shape[0]), lambda i: (0, 0)))
        args.append(w)
        args.append(b.reshape(1, -1))
    out = pl.pallas_call(
        functools.partial(_mlp_chain_body, nlayers, tuple(relus)),
        grid=grid,
        in_specs=in_specs,
        out_specs=pl.BlockSpec((rt, cout), lambda i: (i, 0)),
        out_shape=jax.ShapeDtypeStruct((Rp, cout), jnp.float32),
    )(*args)
    return out[:R]


def _mlp(layers, x):
    shp = x.shape
    r = 1
    for s in shp[:-1]:
        r *= s
    y = _mlp_pallas(x.reshape(r, shp[-1]), layers)
    return y.reshape(shp[:-1] + (y.shape[-1],))


def _conv(layer, x):
    shp = x.shape
    r = 1
    for s in shp[:-1]:
        r *= s
    y = _mlp_pallas(x.reshape(r, shp[-1]), layer, relus=(False,))
    return y.reshape(shp[:-1] + (y.shape[-1],))


def _knn(q, k, n):
    d = (jnp.sum(q * q, -1)[:, :, None] + jnp.sum(k * k, -1)[:, None, :]
         - 2.0 * jnp.einsum('bnd,bmd->bnm', q, k))
    _, idx = jax.lax.top_k(-d, n)
    return idx


def _gather(pts, idx):
    return jax.vmap(lambda p, i: p[i])(pts, idx)


def _sa(p, xyz, label, points, npoint, nsample, xyz_raw=None):
    N = xyz.shape[1]
    sidx = jnp.arange(npoint) * (N // npoint)
    new_xyz = xyz[:, sidx]
    new_label = label[:, sidx]
    nidx = _knn(new_xyz, xyz, nsample)
    g_xyz = _gather(xyz, nidx) - new_xyz[:, :, None, :]
    g_pts = _gather(points, nidx)
    feat = _mlp(p['mlp'], jnp.concatenate([g_xyz, g_pts], -1))
    feat = jnp.max(feat, axis=2)
    feat = _mlp(p['mlp2'], feat)
    if xyz_raw is not None:
        return new_xyz, new_label, feat, xyz_raw[:, sidx]
    return new_xyz, new_label, feat


def _cost(p, xyz1, pts1, xyz2, pts2, nsample, nsample_q):
    qidx = _knn(xyz1, xyz2, nsample_q)
    g_xyz = _gather(xyz2, qidx) - xyz1[:, :, None, :]
    g_pts = _gather(pts2, qidx)
    rep = jnp.broadcast_to(pts1[:, :, None, :], g_pts.shape[:3] + (pts1.shape[-1],))
    feat = _mlp(p['mlp1'], jnp.concatenate([g_pts, rep, g_xyz], -1))
    interim = jnp.max(feat, 2)
    sidx = _knn(xyz1, xyz1, nsample)
    s_xyz = _gather(xyz1, sidx) - xyz1[:, :, None, :]
    s_pts = _gather(interim, sidx)
    feat2 = _mlp(p['mlp2'], jnp.concatenate([s_pts, s_xyz], -1))
    return jnp.max(feat2, 2)


def _upconv(p, xyz1, xyz2, pts1, pts2, nsample):
    nidx = _knn(xyz1, xyz2, nsample)
    g_xyz = _gather(xyz2, nidx) - xyz1[:, :, None, :]
    g_pts = _gather(pts2, nidx)
    feat = _mlp(p['mlp'], jnp.concatenate([g_pts, g_xyz], -1))
    feat = jnp.max(feat, 2)
    return _mlp(p['mlp2'], jnp.concatenate([feat, pts1], -1))


def _fp(p, xyz1, xyz2, pts1, pts2):
    idx = _knn(xyz1, xyz2, 3)
    diff = _gather(xyz2, idx) - xyz1[:, :, None, :]
    dist = jnp.sum(diff * diff, -1)
    w = 1.0 / (dist + 1e-10)
    w = w / jnp.sum(w, -1, keepdims=True)
    interp = jnp.sum(_gather(pts2, idx) * w[..., None], 2)
    if pts1 is not None:
        interp = jnp.concatenate([pts1, interp], -1)
    if len(p['mlp']) > 0:
        interp = _mlp(p['mlp'], interp)
    return interp


def _forward(xyz1, xyz2, color1, color2, label, p):
    xc = jnp.mean(xyz1, 1, keepdims=True)
    x1 = xyz1 - xc
    x2 = xyz2 - xc
    l0x1, l0lab, l0p1, pc1s = _sa(p['sa0'], x1, label, color1, 2048, 32, xyz_raw=xyz1)
    l1x1, l1lab, l1p1 = _sa(p['sa1'], l0x1, l0lab, l0p1, 1024, 24)
    l2x1, l2lab, l2p1 = _sa(p['sa2'], l1x1, l1lab, l1p1, 256, 16)
    l0x2, _, l0p2, pc2s = _sa(p['sa0'], x2, label, color2, 2048, 32, xyz_raw=xyz2)
    l1x2, _, l1p2 = _sa(p['sa1'], l0x2, l0lab, l0p2, 1024, 24)
    l2x2, _, l2p2 = _sa(p['sa2'], l1x2, l1lab, l1p2, 256, 16)
    l3x2, _, l3p2 = _sa(p['sa3_2'], l2x2, l2lab, l2p2, 64, 16)
    l2p1n = _cost(p['cost1'], l2x1, l2p1, l2x2, l2p2, 4, 32)
    l3x1, l3lab, l3p1 = _sa(p['sa3_1'], l2x1, l2lab, l2p1n, 64, 8)
    l4x1, _, l4p1 = _sa(p['sa4_1'], l3x1, l3lab, l3p1, 16, 8)
    l3feat = _upconv(p['up1'], l3x1, l4x1, l3p1, l4p1, 8)
    l3flow_c = _conv(p['conv1'], l3feat)
    l3warp = l3x1 + l3flow_c
    l3cv = _cost(p['cost2'], l3warp, l3p1, l3x2, l3p2, 4, 6)
    l3fine = _mlp(p['pred1']['mlp'], jnp.concatenate([l3p1, l3feat, l3cv], -1))
    l3flow = l3flow_c + _conv(p['conv2'], l3fine)
    l2p1n2 = _upconv(p['up2'], l2x1, l3x1, l2p1, l3fine, 8)
    l2flow_c = _fp(p['fp1'], l2x1, l3x1, None, l3flow)
    l2warp = l2x1 + l2flow_c
    l2cv = _cost(p['cost3'], l2warp, l2p1, l2x2, l2p2, 4, 6)
    l2fine = _mlp(p['pred2']['mlp'], jnp.concatenate([l2p1, l2p1n2, l2cv], -1))
    l2flow = l2flow_c + _conv(p['conv3'], l2fine)
    l1p1n = _upconv(p['up3'], l1x1, l2x1, l1p1, l2fine, 8)
    l1flow_c = _fp(p['fp2'], l1x1, l2x1, None, l2flow)
    l1warp = l1x1 + l1flow_c
    l1cv = _cost(p['cost4'], l1warp, l1p1, l1x2, l1p2, 4, 6)
    l1fine = _mlp(p['pred3']['mlp'], jnp.concatenate([l1p1, l1p1n, l1cv], -1))
    l1flow = l1flow_c + _conv(p['conv4'], l1fine)
    l0feat = _fp(p['fp3'], l0x1, l1x1, l0p1, l1fine)
    net = _conv(p['conv5'], l0feat)
    l0flow_c = _fp(p['fp4'], l0x1, l1x1, None, l1flow)
    l0flow = l0flow_c + _conv(p['conv6'], net)
    return (jnp.transpose(l0flow, (0, 2, 1)),
            jnp.transpose(l1flow, (0, 2, 1)),
            jnp.transpose(l2flow, (0, 2, 1)),
            jnp.transpose(l3flow, (0, 2, 1)))


def kernel(xyz1, xyz2, color1, color2, label, params):
    return _forward(xyz1, xyz2, color1, color2, label, params)


# speedup vs baseline: 1.9531x; 1.9531x over previous
import functools

import jax
import jax.numpy as jnp
from jax.experimental import pallas as pl


def _mlp_chain_body(nlayers, relus, x_ref, *refs):
    out_ref = refs[-1]
    x = x_ref[...]
    for i in range(nlayers):
        w = refs[2 * i][...]
        b = refs[2 * i + 1][...]
        x = jax.lax.dot_general(x, w, (((1,), (0,)), ((), ())),
                                preferred_element_type=jnp.float32)
        x = x + b
        if relus[i]:
            x = jnp.maximum(x, 0.0)
    out_ref[...] = x


def _mlp_pallas(x, layers, relus=None, row_tile=512):
    nlayers = len(layers)
    if relus is None:
        relus = (True,) * nlayers
    R, cin = x.shape
    cout = layers[-1][0].shape[1]
    rt = min(row_tile, R)
    Rp = ((R + rt - 1) // rt) * rt
    if Rp != R:
        x = jnp.pad(x, ((0, Rp - R), (0, 0)))
    grid = (Rp // rt,)
    in_specs = [pl.BlockSpec((rt, cin), lambda i: (i, 0))]
    args = [x]
    for (w, b) in layers:
        in_specs.append(pl.BlockSpec(w.shape, lambda i: (0, 0)))
        in_specs.append(pl.BlockSpec((1, b.shape[0]), lambda i: (0, 0)))
        args.append(w)
        args.append(b.reshape(1, -1))
    out = pl.pallas_call(
        functools.partial(_mlp_chain_body, nlayers, tuple(relus)),
        grid=grid,
        in_specs=in_specs,
        out_specs=pl.BlockSpec((rt, cout), lambda i: (i, 0)),
        out_shape=jax.ShapeDtypeStruct((Rp, cout), jnp.float32),
    )(*args)
    return out[:R]


def _mlp(layers, x):
    shp = x.shape
    r = 1
    for s in shp[:-1]:
        r *= s
    y = _mlp_pallas(x.reshape(r, shp[-1]), layers)
    return y.reshape(shp[:-1] + (y.shape[-1],))


def _conv(layer, x):
    shp = x.shape
    r = 1
    for s in shp[:-1]:
        r *= s
    y = _mlp_pallas(x.reshape(r, shp[-1]), layer, relus=(False,))
    return y.reshape(shp[:-1] + (y.shape[-1],))


_INF = 3.0e38


def _knn_body(nk, TM, N, q_ref, kt_ref, idx_ref):
    q = q_ref[0]
    kt = kt_ref[0]
    qq = jnp.sum(q * q, axis=1, keepdims=True)
    kk = jnp.sum(kt * kt, axis=0, keepdims=True)
    d = qq + kk - 2.0 * jax.lax.dot_general(
        q, kt, (((1,), (0,)), ((), ())), preferred_element_type=jnp.float32)
    iota = jax.lax.broadcasted_iota(jnp.int32, (TM, N), 1)
    cols = []
    for _ in range(nk):
        m = jnp.min(d, axis=1, keepdims=True)
        cand = jnp.where(d == m, iota, jnp.int32(2147483647))
        sel = jnp.min(cand, axis=1, keepdims=True)
        cols.append(sel)
        d = jnp.where(iota == sel, _INF, d)
    idx_ref[0] = jnp.concatenate(cols, axis=1)


def _knn(q, k, n):
    B, M, _ = q.shape
    N = k.shape[1]
    kt = jnp.transpose(k, (0, 2, 1))
    TM = M
    while TM * N * 4 > 8 * 1024 * 1024 and TM % 2 == 0:
        TM //= 2
    grid = (B, M // TM)
    idx = pl.pallas_call(
        functools.partial(_knn_body, n, TM, N),
        grid=grid,
        in_specs=[
            pl.BlockSpec((1, TM, 3), lambda b, i: (b, i, 0)),
            pl.BlockSpec((1, 3, N), lambda b, i: (b, 0, 0)),
        ],
        out_specs=pl.BlockSpec((1, TM, n), lambda b, i: (b, i, 0)),
        out_shape=jax.ShapeDtypeStruct((B, M, n), jnp.int32),
    )(q, kt)
    return idx


def _gather(pts, idx):
    return jax.vmap(lambda p, i: p[i])(pts, idx)


def _sa(p, xyz, label, points, npoint, nsample, xyz_raw=None):
    N = xyz.shape[1]
    sidx = jnp.arange(npoint) * (N // npoint)
    new_xyz = xyz[:, sidx]
    new_label = label[:, sidx]
    nidx = _knn(new_xyz, xyz, nsample)
    g_xyz = _gather(xyz, nidx) - new_xyz[:, :, None, :]
    g_pts = _gather(points, nidx)
    feat = _mlp(p['mlp'], jnp.concatenate([g_xyz, g_pts], -1))
    feat = jnp.max(feat, axis=2)
    feat = _mlp(p['mlp2'], feat)
    if xyz_raw is not None:
        return new_xyz, new_label, feat, xyz_raw[:, sidx]
    return new_xyz, new_label, feat


def _cost(p, xyz1, pts1, xyz2, pts2, nsample, nsample_q):
    qidx = _knn(xyz1, xyz2, nsample_q)
    g_xyz = _gather(xyz2, qidx) - xyz1[:, :, None, :]
    g_pts = _gather(pts2, qidx)
    rep = jnp.broadcast_to(pts1[:, :, None, :], g_pts.shape[:3] + (pts1.shape[-1],))
    feat = _mlp(p['mlp1'], jnp.concatenate([g_pts, rep, g_xyz], -1))
    interim = jnp.max(feat, 2)
    sidx = _knn(xyz1, xyz1, nsample)
    s_xyz = _gather(xyz1, sidx) - xyz1[:, :, None, :]
    s_pts = _gather(interim, sidx)
    feat2 = _mlp(p['mlp2'], jnp.concatenate([s_pts, s_xyz], -1))
    return jnp.max(feat2, 2)


def _upconv(p, xyz1, xyz2, pts1, pts2, nsample):
    nidx = _knn(xyz1, xyz2, nsample)
    g_xyz = _gather(xyz2, nidx) - xyz1[:, :, None, :]
    g_pts = _gather(pts2, nidx)
    feat = _mlp(p['mlp'], jnp.concatenate([g_pts, g_xyz], -1))
    feat = jnp.max(feat, 2)
    return _mlp(p['mlp2'], jnp.concatenate([feat, pts1], -1))


def _fp(p, xyz1, xyz2, pts1, pts2):
    idx = _knn(xyz1, xyz2, 3)
    diff = _gather(xyz2, idx) - xyz1[:, :, None, :]
    dist = jnp.sum(diff * diff, -1)
    w = 1.0 / (dist + 1e-10)
    w = w / jnp.sum(w, -1, keepdims=True)
    interp = jnp.sum(_gather(pts2, idx) * w[..., None], 2)
    if pts1 is not None:
        interp = jnp.concatenate([pts1, interp], -1)
    if len(p['mlp']) > 0:
        interp = _mlp(p['mlp'], interp)
    return interp


def _forward(xyz1, xyz2, color1, color2, label, p):
    xc = jnp.mean(xyz1, 1, keepdims=True)
    x1 = xyz1 - xc
    x2 = xyz2 - xc
    l0x1, l0lab, l0p1, pc1s = _sa(p['sa0'], x1, label, color1, 2048, 32, xyz_raw=xyz1)
    l1x1, l1lab, l1p1 = _sa(p['sa1'], l0x1, l0lab, l0p1, 1024, 24)
    l2x1, l2lab, l2p1 = _sa(p['sa2'], l1x1, l1lab, l1p1, 256, 16)
    l0x2, _, l0p2, pc2s = _sa(p['sa0'], x2, label, color2, 2048, 32, xyz_raw=xyz2)
    l1x2, _, l1p2 = _sa(p['sa1'], l0x2, l0lab, l0p2, 1024, 24)
    l2x2, _, l2p2 = _sa(p['sa2'], l1x2, l1lab, l1p2, 256, 16)
    l3x2, _, l3p2 = _sa(p['sa3_2'], l2x2, l2lab, l2p2, 64, 16)
    l2p1n = _cost(p['cost1'], l2x1, l2p1, l2x2, l2p2, 4, 32)
    l3x1, l3lab, l3p1 = _sa(p['sa3_1'], l2x1, l2lab, l2p1n, 64, 8)
    l4x1, _, l4p1 = _sa(p['sa4_1'], l3x1, l3lab, l3p1, 16, 8)
    l3feat = _upconv(p['up1'], l3x1, l4x1, l3p1, l4p1, 8)
    l3flow_c = _conv(p['conv1'], l3feat)
    l3warp = l3x1 + l3flow_c
    l3cv = _cost(p['cost2'], l3warp, l3p1, l3x2, l3p2, 4, 6)
    l3fine = _mlp(p['pred1']['mlp'], jnp.concatenate([l3p1, l3feat, l3cv], -1))
    l3flow = l3flow_c + _conv(p['conv2'], l3fine)
    l2p1n2 = _upconv(p['up2'], l2x1, l3x1, l2p1, l3fine, 8)
    l2flow_c = _fp(p['fp1'], l2x1, l3x1, None, l3flow)
    l2warp = l2x1 + l2flow_c
    l2cv = _cost(p['cost3'], l2warp, l2p1, l2x2, l2p2, 4, 6)
    l2fine = _mlp(p['pred2']['mlp'], jnp.concatenate([l2p1, l2p1n2, l2cv], -1))
    l2flow = l2flow_c + _conv(p['conv3'], l2fine)
    l1p1n = _upconv(p['up3'], l1x1, l2x1, l1p1, l2fine, 8)
    l1flow_c = _fp(p['fp2'], l1x1, l2x1, None, l2flow)
    l1warp = l1x1 + l1flow_c
    l1cv = _cost(p['cost4'], l1warp, l1p1, l1x2, l1p2, 4, 6)
    l1fine = _mlp(p['pred3']['mlp'], jnp.concatenate([l1p1, l1p1n, l1cv], -1))
    l1flow = l1flow_c + _conv(p['conv4'], l1fine)
    l0feat = _fp(p['fp3'], l0x1, l1x1, l0p1, l1fine)
    net = _conv(p['conv5'], l0feat)
    l0flow_c = _fp(p['fp4'], l0x1, l1x1, None, l1flow)
    l0flow = l0flow_c + _conv(p['conv6'], net)
    return (jnp.transpose(l0flow, (0, 2, 1)),
            jnp.transpose(l1flow, (0, 2, 1)),
            jnp.transpose(l2flow, (0, 2, 1)),
            jnp.transpose(l3flow, (0, 2, 1)))


def kernel(xyz1, xyz2, color1, color2, label, params):
    return _forward(xyz1, xyz2, color1, color2, label, params)


# speedup vs baseline: 5.7015x; 2.9192x over previous
import functools

import jax
import jax.numpy as jnp
from jax.experimental import pallas as pl
from jax.experimental.pallas import tpu as pltpu
from jax.experimental.pallas import tpu_sc as plsc


def _mlp_chain_body(nlayers, relus, x_ref, *refs):
    out_ref = refs[-1]
    x = x_ref[...]
    for i in range(nlayers):
        w = refs[2 * i][...]
        b = refs[2 * i + 1][...]
        x = jax.lax.dot_general(x, w, (((1,), (0,)), ((), ())),
                                preferred_element_type=jnp.float32)
        x = x + b
        if relus[i]:
            x = jnp.maximum(x, 0.0)
    out_ref[...] = x


def _mlp_pallas(x, layers, relus=None, row_tile=512):
    nlayers = len(layers)
    if relus is None:
        relus = (True,) * nlayers
    R, cin = x.shape
    cout = layers[-1][0].shape[1]
    rt = min(row_tile, R)
    Rp = ((R + rt - 1) // rt) * rt
    if Rp != R:
        x = jnp.pad(x, ((0, Rp - R), (0, 0)))
    grid = (Rp // rt,)
    in_specs = [pl.BlockSpec((rt, cin), lambda i: (i, 0))]
    args = [x]
    for (w, b) in layers:
        in_specs.append(pl.BlockSpec(w.shape, lambda i: (0, 0)))
        in_specs.append(pl.BlockSpec((1, b.shape[0]), lambda i: (0, 0)))
        args.append(w)
        args.append(b.reshape(1, -1))
    out = pl.pallas_call(
        functools.partial(_mlp_chain_body, nlayers, tuple(relus)),
        grid=grid,
        in_specs=in_specs,
        out_specs=pl.BlockSpec((rt, cout), lambda i: (i, 0)),
        out_shape=jax.ShapeDtypeStruct((Rp, cout), jnp.float32),
    )(*args)
    return out[:R]


def _mlp(layers, x):
    shp = x.shape
    r = 1
    for s in shp[:-1]:
        r *= s
    y = _mlp_pallas(x.reshape(r, shp[-1]), layers)
    return y.reshape(shp[:-1] + (y.shape[-1],))


def _conv(layer, x):
    shp = x.shape
    r = 1
    for s in shp[:-1]:
        r *= s
    y = _mlp_pallas(x.reshape(r, shp[-1]), layer, relus=(False,))
    return y.reshape(shp[:-1] + (y.shape[-1],))


_INF = 3.0e38


def _knn_body(nk, TM, N, q_ref, kt_ref, idx_ref):
    q = q_ref[0]
    kt = kt_ref[0]
    qq = jnp.sum(q * q, axis=1, keepdims=True)
    kk = jnp.sum(kt * kt, axis=0, keepdims=True)
    d = qq + kk - 2.0 * jax.lax.dot_general(
        q, kt, (((1,), (0,)), ((), ())), preferred_element_type=jnp.float32)
    iota = jax.lax.broadcasted_iota(jnp.int32, (TM, N), 1)
    cols = []
    for _ in range(nk):
        m = jnp.min(d, axis=1, keepdims=True)
        cand = jnp.where(d == m, iota, jnp.int32(2147483647))
        sel = jnp.min(cand, axis=1, keepdims=True)
        cols.append(sel)
        d = jnp.where(iota == sel, _INF, d)
    idx_ref[0] = jnp.concatenate(cols, axis=1)


def _knn(q, k, n):
    B, M, _ = q.shape
    N = k.shape[1]
    kt = jnp.transpose(k, (0, 2, 1))
    TM = M
    while TM * N * 4 > 8 * 1024 * 1024 and TM % 2 == 0:
        TM //= 2
    grid = (B, M // TM)
    idx = pl.pallas_call(
        functools.partial(_knn_body, n, TM, N),
        grid=grid,
        in_specs=[
            pl.BlockSpec((1, TM, 3), lambda b, i: (b, i, 0)),
            pl.BlockSpec((1, 3, N), lambda b, i: (b, 0, 0)),
        ],
        out_specs=pl.BlockSpec((1, TM, n), lambda b, i: (b, i, 0)),
        out_shape=jax.ShapeDtypeStruct((B, M, n), jnp.int32),
    )(q, kt)
    return idx


_NC, _NS = 2, 16
_NW = _NC * _NS


@functools.lru_cache(maxsize=None)
def _build_sc_gather(V, D, Bn, chunk):
    rows_per_w = Bn // _NW
    nchunk = rows_per_w // chunk
    mesh = plsc.VectorSubcoreMesh(core_axis_name="c", subcore_axis_name="s")

    @functools.partial(
        pl.kernel, mesh=mesh,
        out_type=jax.ShapeDtypeStruct((Bn, D), jnp.float32),
        scratch_types=[
            pltpu.VMEM((rows_per_w,), jnp.int32),
            pltpu.VMEM((chunk, D), jnp.float32),
            pltpu.VMEM((chunk, D), jnp.float32),
            pltpu.SemaphoreType.DMA,
            pltpu.SemaphoreType.DMA,
        ],
    )
    def k(table_hbm, idx_hbm, out_hbm, idx_v, buf0, buf1, sem0, sem1):
        wid = jax.lax.axis_index("s") * _NC + jax.lax.axis_index("c")
        base = wid * rows_per_w
        pltpu.sync_copy(idx_hbm.at[pl.ds(base, rows_per_w)], idx_v)
        if nchunk == 1:
            pltpu.async_copy(table_hbm.at[idx_v], buf0, sem0).wait()
            pltpu.sync_copy(buf0, out_hbm.at[pl.ds(base, chunk)])
        else:
            pltpu.async_copy(
                table_hbm.at[idx_v.at[pl.ds(0, chunk)]], buf0, sem0)

            def pair(jj, _):
                ja = jj * 2
                jb = ja + 1

                @pl.when(jb < nchunk)
                def _():
                    pltpu.async_copy(
                        table_hbm.at[idx_v.at[pl.ds(jb * chunk, chunk)]],
                        buf1, sem1)

                pltpu.make_async_copy(
                    table_hbm.at[idx_v.at[pl.ds(ja * chunk, chunk)]],
                    buf0, sem0).wait()
                pltpu.sync_copy(buf0, out_hbm.at[pl.ds(base + ja * chunk, chunk)])

                @pl.when(jb + 1 < nchunk)
                def _():
                    pltpu.async_copy(
                        table_hbm.at[idx_v.at[pl.ds((jb + 1) * chunk, chunk)]],
                        buf0, sem0)

                @pl.when(jb < nchunk)
                def _():
                    pltpu.make_async_copy(
                        table_hbm.at[idx_v.at[pl.ds(jb * chunk, chunk)]],
                        buf1, sem1).wait()
                    pltpu.sync_copy(
                        buf1, out_hbm.at[pl.ds(base + jb * chunk, chunk)])
                return 0

            jax.lax.fori_loop(0, (nchunk + 1) // 2, pair, 0)

    return k


def _gather_rows(table, idxf):
    V, D0 = table.shape
    D = ((D0 + 127) // 128) * 128
    if D != D0:
        table = jnp.pad(table, ((0, 0), (0, D - D0)))
    chunk = 128 if D <= 384 else 64
    unit = _NW * chunk
    Bn0 = idxf.shape[0]
    if Bn0 <= unit:
        Bnp = unit
    else:
        Bnp = ((Bn0 + 2 * unit - 1) // (2 * unit)) * (2 * unit)
    if Bnp != Bn0:
        idxf = jnp.pad(idxf, (0, Bnp - Bn0))
    out = _build_sc_gather(V, D, Bnp, chunk)(table, idxf)
    return out[:Bn0, :D0]


def _gather(pts, idx):
    B, N, C = pts.shape
    _, M, K = idx.shape
    table = pts.reshape(B * N, C)
    idxf = (idx + (jnp.arange(B, dtype=idx.dtype) * N)[:, None, None]).reshape(-1)
    return _gather_rows(table, idxf).reshape(B, M, K, C)


def _sa(p, xyz, label, points, npoint, nsample, xyz_raw=None):
    N = xyz.shape[1]
    sidx = jnp.arange(npoint) * (N // npoint)
    new_xyz = xyz[:, sidx]
    new_label = label[:, sidx]
    nidx = _knn(new_xyz, xyz, nsample)
    g = _gather(jnp.concatenate([xyz, points], -1), nidx)
    feat = _mlp(p['mlp'], jnp.concatenate(
        [g[..., :3] - new_xyz[:, :, None, :], g[..., 3:]], -1))
    feat = jnp.max(feat, axis=2)
    feat = _mlp(p['mlp2'], feat)
    if xyz_raw is not None:
        return new_xyz, new_label, feat, xyz_raw[:, sidx]
    return new_xyz, new_label, feat


def _cost(p, xyz1, pts1, xyz2, pts2, nsample, nsample_q):
    qidx = _knn(xyz1, xyz2, nsample_q)
    g = _gather(jnp.concatenate([xyz2, pts2], -1), qidx)
    g_xyz = g[..., :3] - xyz1[:, :, None, :]
    g_pts = g[..., 3:]
    rep = jnp.broadcast_to(pts1[:, :, None, :], g_pts.shape[:3] + (pts1.shape[-1],))
    feat = _mlp(p['mlp1'], jnp.concatenate([g_pts, rep, g_xyz], -1))
    interim = jnp.max(feat, 2)
    sidx = _knn(xyz1, xyz1, nsample)
    s = _gather(jnp.concatenate([xyz1, interim], -1), sidx)
    s_xyz = s[..., :3] - xyz1[:, :, None, :]
    s_pts = s[..., 3:]
    feat2 = _mlp(p['mlp2'], jnp.concatenate([s_pts, s_xyz], -1))
    return jnp.max(feat2, 2)


def _upconv(p, xyz1, xyz2, pts1, pts2, nsample):
    nidx = _knn(xyz1, xyz2, nsample)
    g = _gather(jnp.concatenate([xyz2, pts2], -1), nidx)
    g_xyz = g[..., :3] - xyz1[:, :, None, :]
    g_pts = g[..., 3:]
    feat = _mlp(p['mlp'], jnp.concatenate([g_pts, g_xyz], -1))
    feat = jnp.max(feat, 2)
    return _mlp(p['mlp2'], jnp.concatenate([feat, pts1], -1))


def _fp(p, xyz1, xyz2, pts1, pts2):
    idx = _knn(xyz1, xyz2, 3)
    g = _gather(jnp.concatenate([xyz2, pts2], -1), idx)
    diff = g[..., :3] - xyz1[:, :, None, :]
    dist = jnp.sum(diff * diff, -1)
    w = 1.0 / (dist + 1e-10)
    w = w / jnp.sum(w, -1, keepdims=True)
    interp = jnp.sum(g[..., 3:] * w[..., None], 2)
    if pts1 is not None:
        interp = jnp.concatenate([pts1, interp], -1)
    if len(p['mlp']) > 0:
        interp = _mlp(p['mlp'], interp)
    return interp


def _forward(xyz1, xyz2, color1, color2, label, p):
    xc = jnp.mean(xyz1, 1, keepdims=True)
    x1 = xyz1 - xc
    x2 = xyz2 - xc
    l0x1, l0lab, l0p1, pc1s = _sa(p['sa0'], x1, label, color1, 2048, 32, xyz_raw=xyz1)
    l1x1, l1lab, l1p1 = _sa(p['sa1'], l0x1, l0lab, l0p1, 1024, 24)
    l2x1, l2lab, l2p1 = _sa(p['sa2'], l1x1, l1lab, l1p1, 256, 16)
    l0x2, _, l0p2, pc2s = _sa(p['sa0'], x2, label, color2, 2048, 32, xyz_raw=xyz2)
    l1x2, _, l1p2 = _sa(p['sa1'], l0x2, l0lab, l0p2, 1024, 24)
    l2x2, _, l2p2 = _sa(p['sa2'], l1x2, l1lab, l1p2, 256, 16)
    l3x2, _, l3p2 = _sa(p['sa3_2'], l2x2, l2lab, l2p2, 64, 16)
    l2p1n = _cost(p['cost1'], l2x1, l2p1, l2x2, l2p2, 4, 32)
    l3x1, l3lab, l3p1 = _sa(p['sa3_1'], l2x1, l2lab, l2p1n, 64, 8)
    l4x1, _, l4p1 = _sa(p['sa4_1'], l3x1, l3lab, l3p1, 16, 8)
    l3feat = _upconv(p['up1'], l3x1, l4x1, l3p1, l4p1, 8)
    l3flow_c = _conv(p['conv1'], l3feat)
    l3warp = l3x1 + l3flow_c
    l3cv = _cost(p['cost2'], l3warp, l3p1, l3x2, l3p2, 4, 6)
    l3fine = _mlp(p['pred1']['mlp'], jnp.concatenate([l3p1, l3feat, l3cv], -1))
    l3flow = l3flow_c + _conv(p['conv2'], l3fine)
    l2p1n2 = _upconv(p['up2'], l2x1, l3x1, l2p1, l3fine, 8)
    l2flow_c = _fp(p['fp1'], l2x1, l3x1, None, l3flow)
    l2warp = l2x1 + l2flow_c
    l2cv = _cost(p['cost3'], l2warp, l2p1, l2x2, l2p2, 4, 6)
    l2fine = _mlp(p['pred2']['mlp'], jnp.concatenate([l2p1, l2p1n2, l2cv], -1))
    l2flow = l2flow_c + _conv(p['conv3'], l2fine)
    l1p1n = _upconv(p['up3'], l1x1, l2x1, l1p1, l2fine, 8)
    l1flow_c = _fp(p['fp2'], l1x1, l2x1, None, l2flow)
    l1warp = l1x1 + l1flow_c
    l1cv = _cost(p['cost4'], l1warp, l1p1, l1x2, l1p2, 4, 6)
    l1fine = _mlp(p['pred3']['mlp'], jnp.concatenate([l1p1, l1p1n, l1cv], -1))
    l1flow = l1flow_c + _conv(p['conv4'], l1fine)
    l0feat = _fp(p['fp3'], l0x1, l1x1, l0p1, l1fine)
    net = _conv(p['conv5'], l0feat)
    l0flow_c = _fp(p['fp4'], l0x1, l1x1, None, l1flow)
    l0flow = l0flow_c + _conv(p['conv6'], net)
    return (jnp.transpose(l0flow, (0, 2, 1)),
            jnp.transpose(l1flow, (0, 2, 1)),
            jnp.transpose(l2flow, (0, 2, 1)),
            jnp.transpose(l3flow, (0, 2, 1)))


def kernel(xyz1, xyz2, color1, color2, label, params):
    return _forward(xyz1, xyz2, color1, color2, label, params)


# speedup vs baseline: 5.7028x; 1.0002x over previous
import functools

import jax
import jax.numpy as jnp
from jax.experimental import pallas as pl
from jax.experimental.pallas import tpu as pltpu
from jax.experimental.pallas import tpu_sc as plsc


def _mlp_chain_body(nlayers, relus, x_ref, *refs):
    out_ref = refs[-1]
    x = x_ref[...]
    for i in range(nlayers):
        w = refs[2 * i][...]
        b = refs[2 * i + 1][...]
        x = jax.lax.dot_general(x, w, (((1,), (0,)), ((), ())),
                                preferred_element_type=jnp.float32)
        x = x + b
        if relus[i]:
            x = jnp.maximum(x, 0.0)
    out_ref[...] = x


def _mlp_pallas(x, layers, relus=None, row_tile=512):
    nlayers = len(layers)
    if relus is None:
        relus = (True,) * nlayers
    R, cin = x.shape
    cout = layers[-1][0].shape[1]
    rt = min(row_tile, R)
    Rp = ((R + rt - 1) // rt) * rt
    if Rp != R:
        x = jnp.pad(x, ((0, Rp - R), (0, 0)))
    grid = (Rp // rt,)
    in_specs = [pl.BlockSpec((rt, cin), lambda i: (i, 0))]
    args = [x]
    for (w, b) in layers:
        in_specs.append(pl.BlockSpec(w.shape, lambda i: (0, 0)))
        in_specs.append(pl.BlockSpec((1, b.shape[0]), lambda i: (0, 0)))
        args.append(w)
        args.append(b.reshape(1, -1))
    out = pl.pallas_call(
        functools.partial(_mlp_chain_body, nlayers, tuple(relus)),
        grid=grid,
        in_specs=in_specs,
        out_specs=pl.BlockSpec((rt, cout), lambda i: (i, 0)),
        out_shape=jax.ShapeDtypeStruct((Rp, cout), jnp.float32),
    )(*args)
    return out[:R]


def _mlp(layers, x):
    shp = x.shape
    r = 1
    for s in shp[:-1]:
        r *= s
    y = _mlp_pallas(x.reshape(r, shp[-1]), layers)
    return y.reshape(shp[:-1] + (y.shape[-1],))


def _conv(layer, x):
    shp = x.shape
    r = 1
    for s in shp[:-1]:
        r *= s
    y = _mlp_pallas(x.reshape(r, shp[-1]), layer, relus=(False,))
    return y.reshape(shp[:-1] + (y.shape[-1],))


_INF = 3.0e38


def _knn_body(nk, TM, N, q_ref, kt_ref, idx_ref):
    q = q_ref[0]
    kt = kt_ref[0]
    qq = jnp.sum(q * q, axis=1, keepdims=True)
    kk = jnp.sum(kt * kt, axis=0, keepdims=True)
    d = qq + kk - 2.0 * jax.lax.dot_general(
        q, kt, (((1,), (0,)), ((), ())), preferred_element_type=jnp.float32)
    iota = jax.lax.broadcasted_iota(jnp.int32, (TM, N), 1)
    cols = []
    for _ in range(nk):
        m = jnp.min(d, axis=1, keepdims=True)
        cand = jnp.where(d == m, iota, jnp.int32(2147483647))
        sel = jnp.min(cand, axis=1, keepdims=True)
        cols.append(sel)
        d = jnp.where(iota == sel, _INF, d)
    idx_ref[0] = jnp.concatenate(cols, axis=1)


def _knn(q, k, n):
    B, M, _ = q.shape
    N = k.shape[1]
    kt = jnp.transpose(k, (0, 2, 1))
    TM = M
    while TM * N * 4 > 8 * 1024 * 1024 and TM % 2 == 0:
        TM //= 2
    grid = (B, M // TM)
    idx = pl.pallas_call(
        functools.partial(_knn_body, n, TM, N),
        grid=grid,
        in_specs=[
            pl.BlockSpec((1, TM, 3), lambda b, i: (b, i, 0)),
            pl.BlockSpec((1, 3, N), lambda b, i: (b, 0, 0)),
        ],
        out_specs=pl.BlockSpec((1, TM, n), lambda b, i: (b, i, 0)),
        out_shape=jax.ShapeDtypeStruct((B, M, n), jnp.int32),
    )(q, kt)
    return idx


_NC, _NS = 2, 16
_NW = _NC * _NS


@functools.lru_cache(maxsize=None)
def _build_sc_gather(V, D, Do, Bn, chunk):
    rows_per_w = Bn // _NW
    nchunk = rows_per_w // chunk
    mesh = plsc.VectorSubcoreMesh(core_axis_name="c", subcore_axis_name="s")

    def _store(buf, dst):
        if Do == D:
            pltpu.sync_copy(buf, dst)
        else:
            pltpu.sync_copy(buf.at[:, pl.ds(0, Do)], dst)

    @functools.partial(
        pl.kernel, mesh=mesh,
        out_type=jax.ShapeDtypeStruct((Bn, Do), jnp.float32),
        scratch_types=[
            pltpu.VMEM((rows_per_w,), jnp.int32),
            pltpu.VMEM((chunk, D), jnp.float32),
            pltpu.VMEM((chunk, D), jnp.float32),
            pltpu.SemaphoreType.DMA,
            pltpu.SemaphoreType.DMA,
        ],
    )
    def k(table_hbm, idx_hbm, out_hbm, idx_v, buf0, buf1, sem0, sem1):
        wid = jax.lax.axis_index("s") * _NC + jax.lax.axis_index("c")
        base = wid * rows_per_w
        pltpu.sync_copy(idx_hbm.at[pl.ds(base, rows_per_w)], idx_v)
        if nchunk == 1:
            pltpu.async_copy(table_hbm.at[idx_v], buf0, sem0).wait()
            _store(buf0, out_hbm.at[pl.ds(base, chunk)])
        else:
            pltpu.async_copy(
                table_hbm.at[idx_v.at[pl.ds(0, chunk)]], buf0, sem0)

            def pair(jj, _):
                ja = jj * 2
                jb = ja + 1

                @pl.when(jb < nchunk)
                def _():
                    pltpu.async_copy(
                        table_hbm.at[idx_v.at[pl.ds(jb * chunk, chunk)]],
                        buf1, sem1)

                pltpu.make_async_copy(
                    table_hbm.at[idx_v.at[pl.ds(ja * chunk, chunk)]],
                    buf0, sem0).wait()
                _store(buf0, out_hbm.at[pl.ds(base + ja * chunk, chunk)])

                @pl.when(jb + 1 < nchunk)
                def _():
                    pltpu.async_copy(
                        table_hbm.at[idx_v.at[pl.ds((jb + 1) * chunk, chunk)]],
                        buf0, sem0)

                @pl.when(jb < nchunk)
                def _():
                    pltpu.make_async_copy(
                        table_hbm.at[idx_v.at[pl.ds(jb * chunk, chunk)]],
                        buf1, sem1).wait()
                    _store(buf1, out_hbm.at[pl.ds(base + jb * chunk, chunk)])
                return 0

            jax.lax.fori_loop(0, (nchunk + 1) // 2, pair, 0)

    return k


def _gather_rows(table, idxf):
    V, D0 = table.shape
    D = ((D0 + 127) // 128) * 128
    if D != D0:
        table = jnp.pad(table, ((0, 0), (0, D - D0)))
    Do = D
    Bn0 = idxf.shape[0]
    if D <= 128 and Bn0 >= 16384:
        chunk = 256
    elif D <= 384:
        chunk = 128
    else:
        chunk = 64
    unit = _NW * chunk
    if Bn0 <= unit:
        Bnp = unit
    else:
        Bnp = ((Bn0 + 2 * unit - 1) // (2 * unit)) * (2 * unit)
    if Bnp != Bn0:
        idxf = jnp.pad(idxf, (0, Bnp - Bn0))
    out = _build_sc_gather(V, D, Do, Bnp, chunk)(table, idxf)
    return out[:Bn0, :D0]


def _gather(pts, idx):
    B, N, C = pts.shape
    _, M, K = idx.shape
    table = pts.reshape(B * N, C)
    idxf = (idx + (jnp.arange(B, dtype=idx.dtype) * N)[:, None, None]).reshape(-1)
    return _gather_rows(table, idxf).reshape(B, M, K, C)


def _sa(p, xyz, label, points, npoint, nsample, xyz_raw=None):
    N = xyz.shape[1]
    sidx = jnp.arange(npoint) * (N // npoint)
    new_xyz = xyz[:, sidx]
    new_label = label[:, sidx]
    nidx = _knn(new_xyz, xyz, nsample)
    g = _gather(jnp.concatenate([xyz, points], -1), nidx)
    feat = _mlp(p['mlp'], jnp.concatenate(
        [g[..., :3] - new_xyz[:, :, None, :], g[..., 3:]], -1))
    feat = jnp.max(feat, axis=2)
    feat = _mlp(p['mlp2'], feat)
    if xyz_raw is not None:
        return new_xyz, new_label, feat, xyz_raw[:, sidx]
    return new_xyz, new_label, feat


def _cost(p, xyz1, pts1, xyz2, pts2, nsample, nsample_q):
    qidx = _knn(xyz1, xyz2, nsample_q)
    g = _gather(jnp.concatenate([xyz2, pts2], -1), qidx)
    g_xyz = g[..., :3] - xyz1[:, :, None, :]
    g_pts = g[..., 3:]
    rep = jnp.broadcast_to(pts1[:, :, None, :], g_pts.shape[:3] + (pts1.shape[-1],))
    feat = _mlp(p['mlp1'], jnp.concatenate([g_pts, rep, g_xyz], -1))
    interim = jnp.max(feat, 2)
    sidx = _knn(xyz1, xyz1, nsample)
    s = _gather(jnp.concatenate([xyz1, interim], -1), sidx)
    s_xyz = s[..., :3] - xyz1[:, :, None, :]
    s_pts = s[..., 3:]
    feat2 = _mlp(p['mlp2'], jnp.concatenate([s_pts, s_xyz], -1))
    return jnp.max(feat2, 2)


def _upconv(p, xyz1, xyz2, pts1, pts2, nsample):
    nidx = _knn(xyz1, xyz2, nsample)
    g = _gather(jnp.concatenate([xyz2, pts2], -1), nidx)
    g_xyz = g[..., :3] - xyz1[:, :, None, :]
    g_pts = g[..., 3:]
    feat = _mlp(p['mlp'], jnp.concatenate([g_pts, g_xyz], -1))
    feat = jnp.max(feat, 2)
    return _mlp(p['mlp2'], jnp.concatenate([feat, pts1], -1))


def _fp(p, xyz1, xyz2, pts1, pts2):
    idx = _knn(xyz1, xyz2, 3)
    g = _gather(jnp.concatenate([xyz2, pts2], -1), idx)
    diff = g[..., :3] - xyz1[:, :, None, :]
    dist = jnp.sum(diff * diff, -1)
    w = 1.0 / (dist + 1e-10)
    w = w / jnp.sum(w, -1, keepdims=True)
    interp = jnp.sum(g[..., 3:] * w[..., None], 2)
    if pts1 is not None:
        interp = jnp.concatenate([pts1, interp], -1)
    if len(p['mlp']) > 0:
        interp = _mlp(p['mlp'], interp)
    return interp


def _forward(xyz1, xyz2, color1, color2, label, p):
    xc = jnp.mean(xyz1, 1, keepdims=True)
    x1 = xyz1 - xc
    x2 = xyz2 - xc
    l0x1, l0lab, l0p1, pc1s = _sa(p['sa0'], x1, label, color1, 2048, 32, xyz_raw=xyz1)
    l1x1, l1lab, l1p1 = _sa(p['sa1'], l0x1, l0lab, l0p1, 1024, 24)
    l2x1, l2lab, l2p1 = _sa(p['sa2'], l1x1, l1lab, l1p1, 256, 16)
    l0x2, _, l0p2, pc2s = _sa(p['sa0'], x2, label, color2, 2048, 32, xyz_raw=xyz2)
    l1x2, _, l1p2 = _sa(p['sa1'], l0x2, l0lab, l0p2, 1024, 24)
    l2x2, _, l2p2 = _sa(p['sa2'], l1x2, l1lab, l1p2, 256, 16)
    l3x2, _, l3p2 = _sa(p['sa3_2'], l2x2, l2lab, l2p2, 64, 16)
    l2p1n = _cost(p['cost1'], l2x1, l2p1, l2x2, l2p2, 4, 32)
    l3x1, l3lab, l3p1 = _sa(p['sa3_1'], l2x1, l2lab, l2p1n, 64, 8)
    l4x1, _, l4p1 = _sa(p['sa4_1'], l3x1, l3lab, l3p1, 16, 8)
    l3feat = _upconv(p['up1'], l3x1, l4x1, l3p1, l4p1, 8)
    l3flow_c = _conv(p['conv1'], l3feat)
    l3warp = l3x1 + l3flow_c
    l3cv = _cost(p['cost2'], l3warp, l3p1, l3x2, l3p2, 4, 6)
    l3fine = _mlp(p['pred1']['mlp'], jnp.concatenate([l3p1, l3feat, l3cv], -1))
    l3flow = l3flow_c + _conv(p['conv2'], l3fine)
    l2p1n2 = _upconv(p['up2'], l2x1, l3x1, l2p1, l3fine, 8)
    l2flow_c = _fp(p['fp1'], l2x1, l3x1, None, l3flow)
    l2warp = l2x1 + l2flow_c
    l2cv = _cost(p['cost3'], l2warp, l2p1, l2x2, l2p2, 4, 6)
    l2fine = _mlp(p['pred2']['mlp'], jnp.concatenate([l2p1, l2p1n2, l2cv], -1))
    l2flow = l2flow_c + _conv(p['conv3'], l2fine)
    l1p1n = _upconv(p['up3'], l1x1, l2x1, l1p1, l2fine, 8)
    l1flow_c = _fp(p['fp2'], l1x1, l2x1, None, l2flow)
    l1warp = l1x1 + l1flow_c
    l1cv = _cost(p['cost4'], l1warp, l1p1, l1x2, l1p2, 4, 6)
    l1fine = _mlp(p['pred3']['mlp'], jnp.concatenate([l1p1, l1p1n, l1cv], -1))
    l1flow = l1flow_c + _conv(p['conv4'], l1fine)
    l0feat = _fp(p['fp3'], l0x1, l1x1, l0p1, l1fine)
    net = _conv(p['conv5'], l0feat)
    l0flow_c = _fp(p['fp4'], l0x1, l1x1, None, l1flow)
    l0flow = l0flow_c + _conv(p['conv6'], net)
    return (jnp.transpose(l0flow, (0, 2, 1)),
            jnp.transpose(l1flow, (0, 2, 1)),
            jnp.transpose(l2flow, (0, 2, 1)),
            jnp.transpose(l3flow, (0, 2, 1)))


def kernel(xyz1, xyz2, color1, color2, label, params):
    return _forward(xyz1, xyz2, color1, color2, label, params)


# speedup vs baseline: 6.0522x; 1.0613x over previous
import functools

import jax
import jax.numpy as jnp
from jax.experimental import pallas as pl
from jax.experimental.pallas import tpu as pltpu
from jax.experimental.pallas import tpu_sc as plsc


def _mlp_chain_body(nlayers, relus, x_ref, *refs):
    out_ref = refs[-1]
    x = x_ref[...]
    for i in range(nlayers):
        w = refs[2 * i][...]
        b = refs[2 * i + 1][...]
        x = jax.lax.dot_general(x, w, (((1,), (0,)), ((), ())),
                                preferred_element_type=jnp.float32)
        x = x + b
        if relus[i]:
            x = jnp.maximum(x, 0.0)
    out_ref[...] = x


def _mlp_pallas(x, layers, relus=None, row_tile=512):
    nlayers = len(layers)
    if relus is None:
        relus = (True,) * nlayers
    R, cin = x.shape
    cout = layers[-1][0].shape[1]
    rt = min(row_tile, R)
    Rp = ((R + rt - 1) // rt) * rt
    if Rp != R:
        x = jnp.pad(x, ((0, Rp - R), (0, 0)))
    grid = (Rp // rt,)
    in_specs = [pl.BlockSpec((rt, cin), lambda i: (i, 0))]
    args = [x]
    for (w, b) in layers:
        in_specs.append(pl.BlockSpec(w.shape, lambda i: (0, 0)))
        in_specs.append(pl.BlockSpec((1, b.shape[0]), lambda i: (0, 0)))
        args.append(w)
        args.append(b.reshape(1, -1))
    out = pl.pallas_call(
        functools.partial(_mlp_chain_body, nlayers, tuple(relus)),
        grid=grid,
        in_specs=in_specs,
        out_specs=pl.BlockSpec((rt, cout), lambda i: (i, 0)),
        out_shape=jax.ShapeDtypeStruct((Rp, cout), jnp.float32),
    )(*args)
    return out[:R]


def _mlp(layers, x):
    shp = x.shape
    r = 1
    for s in shp[:-1]:
        r *= s
    y = _mlp_pallas(x.reshape(r, shp[-1]), layers)
    return y.reshape(shp[:-1] + (y.shape[-1],))


def _conv(layer, x):
    shp = x.shape
    r = 1
    for s in shp[:-1]:
        r *= s
    y = _mlp_pallas(x.reshape(r, shp[-1]), layer, relus=(False,))
    return y.reshape(shp[:-1] + (y.shape[-1],))


_INF = 3.0e38


def _knn_body(nk, TM, N, q_ref, kt_ref, idx_ref):
    q = q_ref[0]
    kt = kt_ref[0]
    qq = jnp.sum(q * q, axis=1, keepdims=True)
    kk = jnp.sum(kt * kt, axis=0, keepdims=True)
    d = qq + kk - 2.0 * jax.lax.dot_general(
        q, kt, (((1,), (0,)), ((), ())), preferred_element_type=jnp.float32)
    iota = jax.lax.broadcasted_iota(jnp.int32, (TM, N), 1)
    cols = []
    for _ in range(nk):
        sel = jnp.argmin(d, axis=1).astype(jnp.int32)[:, None]
        cols.append(sel)
        d = jnp.where(iota == sel, _INF, d)
    idx_ref[0] = jnp.concatenate(cols, axis=1)


def _knn(q, k, n):
    B, M, _ = q.shape
    N = k.shape[1]
    kt = jnp.transpose(k, (0, 2, 1))
    TM = M
    while TM * N * 4 > 8 * 1024 * 1024 and TM % 2 == 0:
        TM //= 2
    grid = (B, M // TM)
    idx = pl.pallas_call(
        functools.partial(_knn_body, n, TM, N),
        grid=grid,
        in_specs=[
            pl.BlockSpec((1, TM, 3), lambda b, i: (b, i, 0)),
            pl.BlockSpec((1, 3, N), lambda b, i: (b, 0, 0)),
        ],
        out_specs=pl.BlockSpec((1, TM, n), lambda b, i: (b, i, 0)),
        out_shape=jax.ShapeDtypeStruct((B, M, n), jnp.int32),
    )(q, kt)
    return idx


_NC, _NS = 2, 16
_NW = _NC * _NS


@functools.lru_cache(maxsize=None)
def _build_sc_gather(V, D, Do, Bn, chunk):
    rows_per_w = Bn // _NW
    nchunk = rows_per_w // chunk
    mesh = plsc.VectorSubcoreMesh(core_axis_name="c", subcore_axis_name="s")

    def _store(buf, dst):
        if Do == D:
            pltpu.sync_copy(buf, dst)
        else:
            pltpu.sync_copy(buf.at[:, pl.ds(0, Do)], dst)

    @functools.partial(
        pl.kernel, mesh=mesh,
        out_type=jax.ShapeDtypeStruct((Bn, Do), jnp.float32),
        scratch_types=[
            pltpu.VMEM((rows_per_w,), jnp.int32),
            pltpu.VMEM((chunk, D), jnp.float32),
            pltpu.VMEM((chunk, D), jnp.float32),
            pltpu.SemaphoreType.DMA,
            pltpu.SemaphoreType.DMA,
        ],
    )
    def k(table_hbm, idx_hbm, out_hbm, idx_v, buf0, buf1, sem0, sem1):
        wid = jax.lax.axis_index("s") * _NC + jax.lax.axis_index("c")
        base = wid * rows_per_w
        pltpu.sync_copy(idx_hbm.at[pl.ds(base, rows_per_w)], idx_v)
        if nchunk == 1:
            pltpu.async_copy(table_hbm.at[idx_v], buf0, sem0).wait()
            _store(buf0, out_hbm.at[pl.ds(base, chunk)])
        else:
            pltpu.async_copy(
                table_hbm.at[idx_v.at[pl.ds(0, chunk)]], buf0, sem0)

            def pair(jj, _):
                ja = jj * 2
                jb = ja + 1

                @pl.when(jb < nchunk)
                def _():
                    pltpu.async_copy(
                        table_hbm.at[idx_v.at[pl.ds(jb * chunk, chunk)]],
                        buf1, sem1)

                pltpu.make_async_copy(
                    table_hbm.at[idx_v.at[pl.ds(ja * chunk, chunk)]],
                    buf0, sem0).wait()
                _store(buf0, out_hbm.at[pl.ds(base + ja * chunk, chunk)])

                @pl.when(jb + 1 < nchunk)
                def _():
                    pltpu.async_copy(
                        table_hbm.at[idx_v.at[pl.ds((jb + 1) * chunk, chunk)]],
                        buf0, sem0)

                @pl.when(jb < nchunk)
                def _():
                    pltpu.make_async_copy(
                        table_hbm.at[idx_v.at[pl.ds(jb * chunk, chunk)]],
                        buf1, sem1).wait()
                    _store(buf1, out_hbm.at[pl.ds(base + jb * chunk, chunk)])
                return 0

            jax.lax.fori_loop(0, (nchunk + 1) // 2, pair, 0)

    return k


def _gather_rows(table, idxf):
    V, D0 = table.shape
    D = ((D0 + 127) // 128) * 128
    if D != D0:
        table = jnp.pad(table, ((0, 0), (0, D - D0)))
    Do = D
    Bn0 = idxf.shape[0]
    if D <= 128 and Bn0 >= 16384:
        chunk = 256
    elif D <= 384:
        chunk = 128
    else:
        chunk = 64
    unit = _NW * chunk
    if Bn0 <= unit:
        Bnp = unit
    else:
        Bnp = ((Bn0 + 2 * unit - 1) // (2 * unit)) * (2 * unit)
    if Bnp != Bn0:
        idxf = jnp.pad(idxf, (0, Bnp - Bn0))
    out = _build_sc_gather(V, D, Do, Bnp, chunk)(table, idxf)
    return out[:Bn0, :D0]


def _gather(pts, idx):
    B, N, C = pts.shape
    _, M, K = idx.shape
    table = pts.reshape(B * N, C)
    idxf = (idx + (jnp.arange(B, dtype=idx.dtype) * N)[:, None, None]).reshape(-1)
    return _gather_rows(table, idxf).reshape(B, M, K, C)


def _sa(p, xyz, label, points, npoint, nsample, xyz_raw=None):
    N = xyz.shape[1]
    sidx = jnp.arange(npoint) * (N // npoint)
    new_xyz = xyz[:, sidx]
    new_label = label[:, sidx]
    nidx = _knn(new_xyz, xyz, nsample)
    g = _gather(jnp.concatenate([xyz, points], -1), nidx)
    feat = _mlp(p['mlp'], jnp.concatenate(
        [g[..., :3] - new_xyz[:, :, None, :], g[..., 3:]], -1))
    feat = jnp.max(feat, axis=2)
    feat = _mlp(p['mlp2'], feat)
    if xyz_raw is not None:
        return new_xyz, new_label, feat, xyz_raw[:, sidx]
    return new_xyz, new_label, feat


def _cost(p, xyz1, pts1, xyz2, pts2, nsample, nsample_q):
    qidx = _knn(xyz1, xyz2, nsample_q)
    g = _gather(jnp.concatenate([xyz2, pts2], -1), qidx)
    g_xyz = g[..., :3] - xyz1[:, :, None, :]
    g_pts = g[..., 3:]
    rep = jnp.broadcast_to(pts1[:, :, None, :], g_pts.shape[:3] + (pts1.shape[-1],))
    feat = _mlp(p['mlp1'], jnp.concatenate([g_pts, rep, g_xyz], -1))
    interim = jnp.max(feat, 2)
    sidx = _knn(xyz1, xyz1, nsample)
    s = _gather(jnp.concatenate([xyz1, interim], -1), sidx)
    s_xyz = s[..., :3] - xyz1[:, :, None, :]
    s_pts = s[..., 3:]
    feat2 = _mlp(p['mlp2'], jnp.concatenate([s_pts, s_xyz], -1))
    return jnp.max(feat2, 2)


def _upconv(p, xyz1, xyz2, pts1, pts2, nsample):
    nidx = _knn(xyz1, xyz2, nsample)
    g = _gather(jnp.concatenate([xyz2, pts2], -1), nidx)
    g_xyz = g[..., :3] - xyz1[:, :, None, :]
    g_pts = g[..., 3:]
    feat = _mlp(p['mlp'], jnp.concatenate([g_pts, g_xyz], -1))
    feat = jnp.max(feat, 2)
    return _mlp(p['mlp2'], jnp.concatenate([feat, pts1], -1))


def _fp(p, xyz1, xyz2, pts1, pts2):
    idx = _knn(xyz1, xyz2, 3)
    g = _gather(jnp.concatenate([xyz2, pts2], -1), idx)
    diff = g[..., :3] - xyz1[:, :, None, :]
    dist = jnp.sum(diff * diff, -1)
    w = 1.0 / (dist + 1e-10)
    w = w / jnp.sum(w, -1, keepdims=True)
    interp = jnp.sum(g[..., 3:] * w[..., None], 2)
    if pts1 is not None:
        interp = jnp.concatenate([pts1, interp], -1)
    if len(p['mlp']) > 0:
        interp = _mlp(p['mlp'], interp)
    return interp


def _forward(xyz1, xyz2, color1, color2, label, p):
    xc = jnp.mean(xyz1, 1, keepdims=True)
    x1 = xyz1 - xc
    x2 = xyz2 - xc
    B = x1.shape[0]
    x12 = jnp.concatenate([x1, x2], 0)
    col12 = jnp.concatenate([color1, color2], 0)
    lab12 = jnp.concatenate([label, label], 0)
    raw12 = jnp.concatenate([xyz1, xyz2], 0)
    l0x12, l0lab12, l0p12, pcs12 = _sa(p['sa0'], x12, lab12, col12, 2048, 32, xyz_raw=raw12)
    l1x12, l1lab12, l1p12 = _sa(p['sa1'], l0x12, l0lab12, l0p12, 1024, 24)
    l2x12, l2lab12, l2p12 = _sa(p['sa2'], l1x12, l1lab12, l1p12, 256, 16)
    l0x1, l0x2 = l0x12[:B], l0x12[B:]
    l0lab = l0lab12[:B]
    l0p1, l0p2 = l0p12[:B], l0p12[B:]
    pc1s, pc2s = pcs12[:B], pcs12[B:]
    l1x1, l1x2 = l1x12[:B], l1x12[B:]
    l1lab = l1lab12[:B]
    l1p1, l1p2 = l1p12[:B], l1p12[B:]
    l2x1, l2x2 = l2x12[:B], l2x12[B:]
    l2lab = l2lab12[:B]
    l2p1, l2p2 = l2p12[:B], l2p12[B:]
    l3x2, _, l3p2 = _sa(p['sa3_2'], l2x2, l2lab, l2p2, 64, 16)
    l2p1n = _cost(p['cost1'], l2x1, l2p1, l2x2, l2p2, 4, 32)
    l3x1, l3lab, l3p1 = _sa(p['sa3_1'], l2x1, l2lab, l2p1n, 64, 8)
    l4x1, _, l4p1 = _sa(p['sa4_1'], l3x1, l3lab, l3p1, 16, 8)
    l3feat = _upconv(p['up1'], l3x1, l4x1, l3p1, l4p1, 8)
    l3flow_c = _conv(p['conv1'], l3feat)
    l3warp = l3x1 + l3flow_c
    l3cv = _cost(p['cost2'], l3warp, l3p1, l3x2, l3p2, 4, 6)
    l3fine = _mlp(p['pred1']['mlp'], jnp.concatenate([l3p1, l3feat, l3cv], -1))
    l3flow = l3flow_c + _conv(p['conv2'], l3fine)
    l2p1n2 = _upconv(p['up2'], l2x1, l3x1, l2p1, l3fine, 8)
    l2flow_c = _fp(p['fp1'], l2x1, l3x1, None, l3flow)
    l2warp = l2x1 + l2flow_c
    l2cv = _cost(p['cost3'], l2warp, l2p1, l2x2, l2p2, 4, 6)
    l2fine = _mlp(p['pred2']['mlp'], jnp.concatenate([l2p1, l2p1n2, l2cv], -1))
    l2flow = l2flow_c + _conv(p['conv3'], l2fine)
    l1p1n = _upconv(p['up3'], l1x1, l2x1, l1p1, l2fine, 8)
    l1flow_c = _fp(p['fp2'], l1x1, l2x1, None, l2flow)
    l1warp = l1x1 + l1flow_c
    l1cv = _cost(p['cost4'], l1warp, l1p1, l1x2, l1p2, 4, 6)
    l1fine = _mlp(p['pred3']['mlp'], jnp.concatenate([l1p1, l1p1n, l1cv], -1))
    l1flow = l1flow_c + _conv(p['conv4'], l1fine)
    l0feat = _fp(p['fp3'], l0x1, l1x1, l0p1, l1fine)
    net = _conv(p['conv5'], l0feat)
    l0flow_c = _fp(p['fp4'], l0x1, l1x1, None, l1flow)
    l0flow = l0flow_c + _conv(p['conv6'], net)
    return (jnp.transpose(l0flow, (0, 2, 1)),
            jnp.transpose(l1flow, (0, 2, 1)),
            jnp.transpose(l2flow, (0, 2, 1)),
            jnp.transpose(l3flow, (0, 2, 1)))


def kernel(xyz1, xyz2, color1, color2, label, params):
    return _forward(xyz1, xyz2, color1, color2, label, params)
